# speedup vs baseline: 2.2552x; 2.2552x over previous
import functools

import jax
import jax.numpy as jnp
from jax import lax
from jax.experimental import pallas as pl
from jax.experimental.pallas import tpu as pltpu
from jax.experimental.pallas import tpu_sc as plsc

B = 4096
D_IN = 512
C = 1000
CPAD = 1024
EMB = 512
K = 10
NEG = -3.0e38

R = 256


def _tc_body(a_ref, w_ref, b_ref, tags_ref, idx_ref):
    logits = jnp.dot(a_ref[...], w_ref[...],
                     preferred_element_type=jnp.float32) + b_ref[...]
    m = jnp.max(logits, axis=-1, keepdims=True)
    e = jnp.exp(logits - m)
    s = jnp.sum(e, axis=-1, keepdims=True)
    tags_ref[...] = (e / s)[:, :C]

    col = lax.broadcasted_iota(jnp.int32, (R, CPAD), 1)
    work = logits
    picks = []
    for _ in range(K):
        mx = jnp.max(work, axis=-1, keepdims=True)
        am = jnp.min(jnp.where(work == mx, col, CPAD), axis=-1, keepdims=True)
        picks.append(am)
        work = jnp.where(col == am, NEG, work)
    idx_ref[...] = jnp.concatenate(picks, axis=1)


def _tc_call(a, w_pad, b_pad):
    return pl.pallas_call(
        _tc_body,
        grid=(B // R,),
        in_specs=[
            pl.BlockSpec((R, D_IN), lambda i: (i, 0)),
            pl.BlockSpec((D_IN, CPAD), lambda i: (0, 0)),
            pl.BlockSpec((1, CPAD), lambda i: (0, 0)),
        ],
        out_specs=[
            pl.BlockSpec((R, C), lambda i: (i, 0)),
            pl.BlockSpec((R, K), lambda i: (i, 0)),
        ],
        out_shape=[
            jax.ShapeDtypeStruct((B, C), jnp.float32),
            jax.ShapeDtypeStruct((B, K), jnp.int32),
        ],
    )(a, w_pad, b_pad)


_NC = 2
_NS = 16
NW = _NC * _NS
TOTAL = B * K
PER_W = TOTAL // NW
CHUNK = 128
NCH = PER_W // CHUNK

_sc_mesh = plsc.VectorSubcoreMesh(core_axis_name="c", subcore_axis_name="s")


@functools.partial(
    pl.kernel,
    mesh=_sc_mesh,
    out_type=jax.ShapeDtypeStruct((TOTAL, EMB), jnp.float32),
    scratch_types=[
        pltpu.VMEM((PER_W,), jnp.int32),
        pltpu.VMEM((CHUNK, EMB), jnp.float32),
        pltpu.SemaphoreType.DMA,
    ],
)
def _sc_gather(table_hbm, idx_hbm, out_hbm, idx_v, rows_v, sem):
    wid = lax.axis_index("s") * _NC + lax.axis_index("c")
    base = wid * PER_W
    pltpu.sync_copy(idx_hbm.at[pl.ds(base, PER_W)], idx_v)
    for c in range(NCH):
        pltpu.async_copy(
            table_hbm.at[idx_v.at[pl.ds(c * CHUNK, CHUNK)]], rows_v, sem
        ).wait()
        pltpu.sync_copy(rows_v, out_hbm.at[pl.ds(base + c * CHUNK, CHUNK)])


def kernel(avg_features, W, b, embed_table):
    w_pad = jnp.pad(W, ((0, 0), (0, CPAD - C)))
    b_pad = jnp.pad(b, (0, CPAD - C), constant_values=NEG).reshape(1, CPAD)
    tags, idx = _tc_call(avg_features, w_pad, b_pad)
    rows = _sc_gather(embed_table, idx.reshape(TOTAL))
    return tags, rows.reshape(B, K, EMB)


# speedup vs baseline: 3.2114x; 1.4240x over previous
import functools

import jax
import jax.numpy as jnp
from jax import lax
from jax.experimental import pallas as pl
from jax.experimental.pallas import tpu as pltpu
from jax.experimental.pallas import tpu_sc as plsc

B = 4096
D_IN = 512
C = 1000
CPAD = 1024
EMB = 512
K = 10
IDXW = 128
NEG = -3.0e38

R = 256


def _tc_body(a_ref, w_ref, b_ref, tags_ref, idx_ref):
    logits = jnp.dot(a_ref[...], w_ref[...],
                     preferred_element_type=jnp.float32) + b_ref[...]
    m = jnp.max(logits, axis=-1, keepdims=True)
    e = jnp.exp(logits - m)
    s = jnp.sum(e, axis=-1, keepdims=True)
    tags_ref[...] = (e / s)[:, :C]

    col = lax.broadcasted_iota(jnp.int32, (R, CPAD), 1)
    work = logits
    picks = []
    for _ in range(K):
        mx = jnp.max(work, axis=-1, keepdims=True)
        am = jnp.min(jnp.where(work == mx, col, CPAD), axis=-1, keepdims=True)
        picks.append(am)
        work = jnp.where(col == am, NEG, work)
    picks.append(jnp.zeros((R, IDXW - K), jnp.int32))
    idx_ref[...] = jnp.concatenate(picks, axis=1)


def _tc_call(a, w_pad, b_pad):
    return pl.pallas_call(
        _tc_body,
        grid=(B // R,),
        in_specs=[
            pl.BlockSpec((R, D_IN), lambda i: (i, 0)),
            pl.BlockSpec((D_IN, CPAD), lambda i: (0, 0)),
            pl.BlockSpec((1, CPAD), lambda i: (0, 0)),
        ],
        out_specs=[
            pl.BlockSpec((R, C), lambda i: (i, 0)),
            pl.BlockSpec((R, IDXW), lambda i: (i, 0)),
        ],
        out_shape=[
            jax.ShapeDtypeStruct((B, C), jnp.float32),
            jax.ShapeDtypeStruct((B, IDXW), jnp.int32),
        ],
    )(a, w_pad, b_pad)


_NC = 2
_NS = 16
NW = _NC * _NS
BPW = B // NW
NB = 8
NCH = BPW // NB

_sc_mesh = plsc.VectorSubcoreMesh(core_axis_name="c", subcore_axis_name="s")

NBUF = 4
KA = 8
KB = K - KA


@functools.partial(
    pl.kernel,
    mesh=_sc_mesh,
    out_type=jax.ShapeDtypeStruct((B, K, EMB), jnp.float32),
    scratch_types=[pltpu.VMEM((BPW, IDXW), jnp.int32)]
    + [pltpu.VMEM((KA, EMB), jnp.float32)] * NBUF
    + [pltpu.VMEM((KB, EMB), jnp.float32)] * NBUF
    + [pltpu.SemaphoreType.DMA] * (4 * NBUF),
)
def _sc_gather(table_hbm, idx_hbm, out_hbm, idx_v, *bufs):
    rows8 = bufs[:NBUF]
    rows2 = bufs[NBUF:2 * NBUF]
    gsem8 = bufs[2 * NBUF:3 * NBUF]
    gsem2 = bufs[3 * NBUF:4 * NBUF]
    osem8 = bufs[4 * NBUF:5 * NBUF]
    osem2 = bufs[5 * NBUF:]
    wid = lax.axis_index("s") * _NC + lax.axis_index("c")
    base = wid * BPW
    pltpu.sync_copy(idx_hbm.at[pl.ds(base, BPW)], idx_v)

    @pl.loop(0, BPW, step=NBUF)
    def _chunk(j):
        for t in range(NBUF):

            @pl.when(j > 0)
            def _():
                bprev = base + j + t - NBUF
                pltpu.make_async_copy(
                    rows8[t], out_hbm.at[bprev, pl.ds(0, KA)], osem8[t]).wait()
                pltpu.make_async_copy(
                    rows2[t], out_hbm.at[bprev, pl.ds(KA, KB)], osem2[t]).wait()

            pltpu.async_copy(
                table_hbm.at[idx_v.at[j + t, pl.ds(0, KA)]],
                rows8[t], gsem8[t])
            pltpu.async_copy(
                table_hbm.at[idx_v.at[j + t, pl.ds(KA, KB)]],
                rows2[t], gsem2[t])
        for t in range(NBUF):
            pltpu.make_async_copy(
                table_hbm.at[idx_v.at[0, pl.ds(0, KA)]],
                rows8[t], gsem8[t]).wait()
            pltpu.make_async_copy(
                table_hbm.at[idx_v.at[0, pl.ds(KA, KB)]],
                rows2[t], gsem2[t]).wait()
            bloc = base + j + t
            pltpu.async_copy(rows8[t], out_hbm.at[bloc, pl.ds(0, KA)], osem8[t])
            pltpu.async_copy(rows2[t], out_hbm.at[bloc, pl.ds(KA, KB)], osem2[t])

    for t in range(NBUF):
        blast = base + BPW - NBUF + t
        pltpu.make_async_copy(
            rows8[t], out_hbm.at[blast, pl.ds(0, KA)], osem8[t]).wait()
        pltpu.make_async_copy(
            rows2[t], out_hbm.at[blast, pl.ds(KA, KB)], osem2[t]).wait()


def kernel(avg_features, W, b, embed_table):
    w_pad = jnp.pad(W, ((0, 0), (0, CPAD - C)))
    b_pad = jnp.pad(b, (0, CPAD - C), constant_values=NEG).reshape(1, CPAD)
    tags, idx = _tc_call(avg_features, w_pad, b_pad)
    semantic = _sc_gather(embed_table, idx)
    return tags, semantic


# speedup vs baseline: 4.6120x; 1.4361x over previous
import functools

import jax
import jax.numpy as jnp
from jax import lax
from jax.experimental import pallas as pl
from jax.experimental.pallas import tpu as pltpu
from jax.experimental.pallas import tpu_sc as plsc

B = 4096
D_IN = 512
C = 1000
CPAD = 1024
EMB = 512
K = 10
KPAD = 16
NEG = -3.0e38

R = 256


def _tc_body(wt_ref, at_ref, b_ref, tags_ref, idx_ref):
    logits = jnp.dot(wt_ref[...], at_ref[...],
                     preferred_element_type=jnp.float32) + b_ref[...]
    m = jnp.max(logits, axis=0, keepdims=True)
    e = jnp.exp(logits - m)
    s = jnp.sum(e, axis=0, keepdims=True)
    tags_ref[...] = (e / s)[:C, :]

    row = lax.broadcasted_iota(jnp.int32, (CPAD, R), 0)
    work = logits
    picks = []
    for _ in range(K):
        mx = jnp.max(work, axis=0, keepdims=True)
        am = jnp.min(jnp.where(work == mx, row, CPAD), axis=0, keepdims=True)
        picks.append(am)
        work = jnp.where(row == am, NEG, work)
    picks.append(jnp.zeros((KPAD - K, R), jnp.int32))
    idx_ref[...] = jnp.concatenate(picks, axis=0)


def _tc_call(wt_pad, at, bt_pad):
    return pl.pallas_call(
        _tc_body,
        grid=(B // R,),
        in_specs=[
            pl.BlockSpec((CPAD, D_IN), lambda i: (0, 0)),
            pl.BlockSpec((D_IN, R), lambda i: (0, i)),
            pl.BlockSpec((CPAD, 1), lambda i: (0, 0)),
        ],
        out_specs=[
            pl.BlockSpec((C, R), lambda i: (0, i)),
            pl.BlockSpec((KPAD, R), lambda i: (0, i)),
        ],
        out_shape=[
            jax.ShapeDtypeStruct((C, B), jnp.float32),
            jax.ShapeDtypeStruct((KPAD, B), jnp.int32),
        ],
    )(wt_pad, at, bt_pad)


_NC = 2
_NS = 16
NW = _NC * _NS
BPW = B // NW
GCH = 64
NBUF = 2

_sc_mesh = plsc.VectorSubcoreMesh(core_axis_name="c", subcore_axis_name="s")


@functools.partial(
    pl.kernel,
    mesh=_sc_mesh,
    out_type=jax.ShapeDtypeStruct((K, B, EMB), jnp.float32),
    scratch_types=[pltpu.VMEM((KPAD, BPW), jnp.int32)]
    + [pltpu.VMEM((GCH, EMB), jnp.float32)] * NBUF
    + [pltpu.SemaphoreType.DMA] * (2 * NBUF),
)
def _sc_gather(table_hbm, idx_hbm, out_hbm, idx_v, *bufs):
    rows = bufs[:NBUF]
    gsem = bufs[NBUF:2 * NBUF]
    osem = bufs[2 * NBUF:]
    wid = lax.axis_index("s") * _NC + lax.axis_index("c")
    base = wid * BPW
    pltpu.sync_copy(idx_hbm.at[pl.ds(0, KPAD), pl.ds(base, BPW)], idx_v)

    nch = K * (BPW // GCH)

    @pl.loop(0, nch, step=NBUF)
    def _chunk(j):
        for t in range(NBUF):
            u = j + t
            k = u // (BPW // GCH)
            half = u % (BPW // GCH)

            @pl.when(j > 0)
            def _():
                up = u - NBUF
                pltpu.make_async_copy(
                    rows[t],
                    out_hbm.at[up // (BPW // GCH),
                               pl.ds(base + (up % (BPW // GCH)) * GCH, GCH)],
                    osem[t]).wait()

            pltpu.async_copy(
                table_hbm.at[idx_v.at[k, pl.ds(half * GCH, GCH)]],
                rows[t], gsem[t])
        for t in range(NBUF):
            u = j + t
            k = u // (BPW // GCH)
            half = u % (BPW // GCH)
            pltpu.make_async_copy(
                table_hbm.at[idx_v.at[0, pl.ds(0, GCH)]],
                rows[t], gsem[t]).wait()
            pltpu.async_copy(
                rows[t], out_hbm.at[k, pl.ds(base + half * GCH, GCH)],
                osem[t])

    for t in range(NBUF):
        ul = nch - NBUF + t
        pltpu.make_async_copy(
            rows[t],
            out_hbm.at[ul // (BPW // GCH),
                       pl.ds(base + (ul % (BPW // GCH)) * GCH, GCH)],
            osem[t]).wait()


def kernel(avg_features, W, b, embed_table):
    wt_pad = jnp.pad(W.T, ((0, CPAD - C), (0, 0)))
    bt_pad = jnp.pad(b, (0, CPAD - C), constant_values=NEG).reshape(CPAD, 1)
    tags_t, idx_t = _tc_call(wt_pad, avg_features.T, bt_pad)
    out_km = _sc_gather(embed_table, idx_t)
    return tags_t.T, out_km.transpose(1, 0, 2)


# speedup vs baseline: 4.6503x; 1.0083x over previous
import functools

import jax
import jax.numpy as jnp
from jax import lax
from jax.experimental import pallas as pl
from jax.experimental.pallas import tpu as pltpu
from jax.experimental.pallas import tpu_sc as plsc

B = 4096
D_IN = 512
C = 1000
CPAD = 1024
EMB = 512
K = 10
KPAD = 16
NEG = -3.0e38

R = 256


def _tc_body(wt_ref, at_ref, b_ref, tags_ref, idx_ref):
    logits = jnp.dot(wt_ref[...], at_ref[...],
                     preferred_element_type=jnp.float32) + b_ref[...]
    m = jnp.max(logits, axis=0, keepdims=True)
    e = jnp.exp(logits - m)
    s = jnp.sum(e, axis=0, keepdims=True)
    tags_ref[...] = (e / s)[:C, :]

    row = lax.broadcasted_iota(jnp.int32, (CPAD, R), 0)
    work = logits
    picks = []
    for _ in range(K):
        mx = jnp.max(work, axis=0, keepdims=True)
        am = jnp.min(jnp.where(work == mx, row, CPAD), axis=0, keepdims=True)
        picks.append(am)
        work = jnp.where(row == am, NEG, work)
    picks.append(jnp.zeros((KPAD - K, R), jnp.int32))
    idx_ref[...] = jnp.concatenate(picks, axis=0)


def _tc_call(wt_pad, at, bt_pad):
    return pl.pallas_call(
        _tc_body,
        grid=(B // R,),
        in_specs=[
            pl.BlockSpec((CPAD, D_IN), lambda i: (0, 0)),
            pl.BlockSpec((D_IN, R), lambda i: (0, i)),
            pl.BlockSpec((CPAD, 1), lambda i: (0, 0)),
        ],
        out_specs=[
            pl.BlockSpec((C, R), lambda i: (0, i)),
            pl.BlockSpec((KPAD, R), lambda i: (0, i)),
        ],
        out_shape=[
            jax.ShapeDtypeStruct((C, B), jnp.float32),
            jax.ShapeDtypeStruct((KPAD, B), jnp.int32),
        ],
    )(wt_pad, at, bt_pad)


_NC = 2
_NS = 16
NW = _NC * _NS
BPW = B // NW
GCH = 32
NBUF = 4

_sc_mesh = plsc.VectorSubcoreMesh(core_axis_name="c", subcore_axis_name="s")


@functools.partial(
    pl.kernel,
    mesh=_sc_mesh,
    out_type=jax.ShapeDtypeStruct((K, B, EMB), jnp.float32),
    scratch_types=[pltpu.VMEM((KPAD, BPW), jnp.int32)]
    + [pltpu.VMEM((GCH, EMB), jnp.float32)] * NBUF
    + [pltpu.SemaphoreType.DMA] * (2 * NBUF),
)
def _sc_gather(table_hbm, idx_hbm, out_hbm, idx_v, *bufs):
    rows = bufs[:NBUF]
    gsem = bufs[NBUF:2 * NBUF]
    osem = bufs[2 * NBUF:]
    wid = lax.axis_index("s") * _NC + lax.axis_index("c")
    base = wid * BPW
    pltpu.sync_copy(idx_hbm.at[pl.ds(0, KPAD), pl.ds(base, BPW)], idx_v)

    nch = K * (BPW // GCH)

    @pl.loop(0, nch, step=NBUF)
    def _chunk(j):
        for t in range(NBUF):
            u = j + t
            k = u // (BPW // GCH)
            half = u % (BPW // GCH)

            @pl.when(j > 0)
            def _():
                up = u - NBUF
                pltpu.make_async_copy(
                    rows[t],
                    out_hbm.at[up // (BPW // GCH),
                               pl.ds(base + (up % (BPW // GCH)) * GCH, GCH)],
                    osem[t]).wait()

            pltpu.async_copy(
                table_hbm.at[idx_v.at[k, pl.ds(half * GCH, GCH)]],
                rows[t], gsem[t])
        for t in range(NBUF):
            u = j + t
            k = u // (BPW // GCH)
            half = u % (BPW // GCH)
            pltpu.make_async_copy(
                table_hbm.at[idx_v.at[0, pl.ds(0, GCH)]],
                rows[t], gsem[t]).wait()
            pltpu.async_copy(
                rows[t], out_hbm.at[k, pl.ds(base + half * GCH, GCH)],
                osem[t])

    for t in range(NBUF):
        ul = nch - NBUF + t
        pltpu.make_async_copy(
            rows[t],
            out_hbm.at[ul // (BPW // GCH),
                       pl.ds(base + (ul % (BPW // GCH)) * GCH, GCH)],
            osem[t]).wait()


def kernel(avg_features, W, b, embed_table):
    wt_pad = jnp.pad(W.T, ((0, CPAD - C), (0, 0)))
    bt_pad = jnp.pad(b, (0, CPAD - C), constant_values=NEG).reshape(CPAD, 1)
    tags_t, idx_t = _tc_call(wt_pad, avg_features.T, bt_pad)
    out_km = _sc_gather(embed_table, idx_t)
    return tags_t.T, out_km.transpose(1, 0, 2)
